# baseline (speedup 1.0000x reference)
import functools

import jax
import jax.numpy as jnp
from jax import lax
from jax.experimental import pallas as pl
from jax.experimental.pallas import tpu as pltpu
from jax.experimental.pallas import tpu_sc as plsc

NC, NS = 2, 16
NW = NC * NS
CHUNK = 128
NBUF = 4


def _make_gather(n, d):
    assert n % (NW * CHUNK) == 0
    per_w = n // NW
    steps = per_w // CHUNK
    assert steps % NBUF == 0
    mesh = plsc.VectorSubcoreMesh(core_axis_name="c", subcore_axis_name="s")

    @functools.partial(
        pl.kernel,
        out_type=jax.ShapeDtypeStruct((n, d), jnp.float32),
        mesh=mesh,
        scratch_types=[
            pltpu.VMEM((steps, CHUNK), jnp.int32),
            pltpu.VMEM((NBUF, CHUNK, d), jnp.float32),
            pltpu.SemaphoreType.DMA((NBUF,)),
        ],
        compiler_params=pltpu.CompilerParams(use_tc_tiling_on_sc=False),
    )
    def gather_kernel(idx_hbm, table_hbm, out_hbm, idx_v, rows_v, gsem):
        wid = lax.axis_index("s") * NC + lax.axis_index("c")
        pltpu.sync_copy(idx_hbm.at[pl.ds(wid * steps, steps)], idx_v)

        for b in range(NBUF):
            pltpu.async_copy(table_hbm.at[idx_v.at[b]], rows_v.at[b],
                             gsem.at[b])

        @pl.loop(0, steps, step=NBUF)
        def step(j):
            for b in range(NBUF):
                i = j + b
                pltpu.make_async_copy(table_hbm.at[idx_v.at[i]],
                                      rows_v.at[b], gsem.at[b]).wait()
                out_base = (wid * steps + i) * CHUNK
                pltpu.sync_copy(rows_v.at[b],
                                out_hbm.at[pl.ds(out_base, CHUNK)])

                @pl.when(i + NBUF < steps)
                def _refire():
                    pltpu.async_copy(table_hbm.at[idx_v.at[i + NBUF]],
                                     rows_v.at[b], gsem.at[b])

    return gather_kernel


def kernel(inputs, table):
    bsz, seq = inputs.shape
    _, d = table.shape
    n = bsz * seq
    idx2d = inputs.astype(jnp.int32).reshape(n // CHUNK, CHUNK)
    out = _make_gather(n, d)(idx2d, table)
    return out.reshape(bsz, seq, d)
